# baseline (speedup 1.0000x reference)
import functools

import jax
import jax.numpy as jnp
from jax.experimental import pallas as pl
from jax.experimental.pallas import tpu as pltpu
from jax.experimental.pallas import tpu_sc as plsc

N = 8192
K = 16
DIMS = (32, 64, 128)
NUM_CLASSES = 40
INF = 1e10
BIG = 3e38

_INTERPRET = False


def _bn(h):
    mu = jnp.mean(h, axis=0, keepdims=True)
    var = jnp.mean((h - mu) ** 2, axis=0, keepdims=True)
    return (h - mu) / jnp.sqrt(var + 1e-5)


def _dot(a, b):
    return jax.lax.dot_general(a, b, (((1,), (0,)), ((), ())),
                               preferred_element_type=jnp.float32)


def _prep0_body(x_ref, pos_ref, inW_ref, inb_ref, Win_ref, bin_ref,
                Wsrc_ref, Wdst_ref, Wlin_ref, adst_ref, pack_ref):
    h = _dot(x_ref[...], inW_ref[...]) + inb_ref[...]
    x0 = jax.nn.relu(_bn(h))
    x1 = jax.nn.relu(_dot(x0, Win_ref[...]) + bin_ref[...])
    a_src = _dot(x1, Wsrc_ref[...])
    adst_ref[...] = _dot(x1, Wdst_ref[...])
    v = _dot(x1, Wlin_ref[...])
    c = a_src.shape[1]
    pack_ref[:, 0:c] = a_src
    pack_ref[:, c:2 * c] = v
    pack_ref[:, 2 * c:2 * c + 3] = pos_ref[...]
    pack_ref[:, 2 * c + 3:] = jnp.zeros_like(pack_ref[:, 2 * c + 3:])


def _pack_width(c):
    return -((2 * c + 3) // -128) * 128


def _prep0(x, pos, params):
    p = params['blk0']
    c = p['Win'].shape[0]
    w = _pack_width(c)
    return pl.pallas_call(
        _prep0_body,
        out_shape=(jax.ShapeDtypeStruct((x.shape[0], c), jnp.float32),
                   jax.ShapeDtypeStruct((x.shape[0], w), jnp.float32)),
        interpret=_INTERPRET,
    )(x, pos, params['in_W'], params['in_b'][None, :], p['Win'],
      p['bin'][None, :], p['Wsrc'], p['Wdst'], p['Wlin'])


def _knn_body(q_ref, sT_ref, out_ref, *, exclude_self, tr, ns):
    q = q_ref[...]
    sT = sT_ref[...]
    qs = jnp.sum(q * q, axis=1, keepdims=True)
    ss = jnp.sum(sT * sT, axis=0, keepdims=True)
    d = qs - 2.0 * _dot(q, sT) + ss
    colv = jax.lax.broadcasted_iota(jnp.int32, (tr, ns), 1)
    if exclude_self:
        row0 = pl.program_id(0) * tr
        rowv = row0 + jax.lax.broadcasted_iota(jnp.int32, (tr, ns), 0)
        d = jnp.where(colv == rowv, INF, d)
    for j in range(K):
        idx = jnp.argmin(d, axis=1).astype(jnp.int32)[:, None]
        out_ref[:, j:j + 1] = idx
        d = jnp.where(colv == idx, BIG, d)


def _knn(qpos_pad, sposT_pad, exclude_self, tr=128):
    nq = qpos_pad.shape[0]
    ns = sposT_pad.shape[1]
    body = functools.partial(_knn_body, exclude_self=exclude_self, tr=tr, ns=ns)
    return pl.pallas_call(
        body,
        grid=(nq // tr,),
        in_specs=[pl.BlockSpec((tr, 8), lambda t: (t, 0)),
                  pl.BlockSpec((8, ns), lambda t: (0, 0))],
        out_specs=pl.BlockSpec((tr, K), lambda t: (t, 0)),
        out_shape=jax.ShapeDtypeStruct((nq, K), jnp.int32),
        interpret=_INTERPRET,
    )(qpos_pad, sposT_pad)


def _fps_body(pos_ref, psub_ref, dist_ref, *, npts, rows):
    px = pos_ref[0]
    py = pos_ref[1]
    pz = pos_ref[2]
    lane3 = jax.lax.broadcasted_iota(jnp.int32, (1, 128), 1)
    rowi = jax.lax.broadcasted_iota(jnp.int32, (rows, 1), 0)

    def extract(r, lmsk):
        cx = jnp.sum(jnp.where(lmsk, pos_ref[0, pl.ds(r, 1), :], 0.0))
        cy = jnp.sum(jnp.where(lmsk, pos_ref[1, pl.ds(r, 1), :], 0.0))
        cz = jnp.sum(jnp.where(lmsk, pos_ref[2, pl.ds(r, 1), :], 0.0))
        return cx, cy, cz

    def store_row(i, cx, cy, cz):
        row = jnp.where(lane3 == 0, cx,
                        jnp.where(lane3 == 1, cy,
                                  jnp.where(lane3 == 2, cz, 0.0)))
        psub_ref[pl.ds(i, 1), :] = row[:, 0:psub_ref.shape[1]]

    cx, cy, cz = extract(0, lane3 == 0)
    d0 = (px - cx) ** 2 + (py - cy) ** 2 + (pz - cz) ** 2
    dist_ref[...] = d0
    store_row(0, cx, cy, cz)

    flati = (jax.lax.broadcasted_iota(jnp.int32, (rows, 128), 0) * 128
             + jax.lax.broadcasted_iota(jnp.int32, (rows, 128), 1))

    def body(i, _):
        dist = dist_ref[...]
        m = jnp.max(dist)
        idx = jnp.min(jnp.where(dist == m, flati, rows * 128))
        r = idx // 128
        lmsk = lane3 == (idx % 128)
        cx, cy, cz = extract(r, lmsk)
        store_row(i, cx, cy, cz)
        d = (px - cx) ** 2 + (py - cy) ** 2 + (pz - cz) ** 2
        dist_ref[...] = jnp.minimum(dist, d)
        return 0

    jax.lax.fori_loop(1, npts, body, 0)


def _fps(pos3, npts):
    rows = pos3.shape[1]
    body = functools.partial(_fps_body, npts=npts, rows=rows)
    return pl.pallas_call(
        body,
        out_shape=jax.ShapeDtypeStruct((npts, 8), jnp.float32),
        scratch_shapes=[pltpu.VMEM((rows, 128), jnp.float32)],
        interpret=_INTERPRET,
    )(pos3)


def _sc_gather_rows(data, idx_flat, window):
    e = idx_flat.shape[0]
    w = data.shape[1]
    idx2 = idx_flat.reshape(1, e)
    mesh = plsc.VectorSubcoreMesh(core_axis_name="c", subcore_axis_name="s")

    @functools.partial(
        pl.kernel,
        out_type=jax.ShapeDtypeStruct((e, w), data.dtype),
        mesh=mesh)
    def kern(x_hbm, i_hbm, o_hbm):
        def body(i_vmem, o_vmem):
            pltpu.sync_copy(x_hbm.at[i_vmem.at[0]], o_vmem)

        pltpu.emit_pipeline(
            body,
            grid=(e // window,),
            in_specs=[pl.BlockSpec((1, window), lambda i: (0, i))],
            out_specs=[pl.BlockSpec((window, w), lambda i: (i, 0))],
            core_axis_name=("c", "s"),
            dimension_semantics=(pltpu.PARALLEL,),
        )(i_hbm, o_hbm)

    return kern(data, idx2)


def _gather_rows(data, idx_flat, window):
    return _sc_gather_rows(data, idx_flat, window)


def _edge_body(G_ref, adst_ref, pos_ref, P1_ref, p1b_ref, P2_ref, p2b_ref,
               A1_ref, a1b_ref, A2_ref, a2b_ref, Wout_ref, bout_ref, out_ref,
               *, c, tr):
    G = G_ref[...]
    a_src = G[:, :, 0:c]
    v = G[:, :, c:2 * c]
    posj = G[:, :, 2 * c:2 * c + 3]
    posi = pos_ref[...][:, 0:3]
    pd = (posi[None, :, :] - posj).reshape(K * tr, 3)
    h1 = jax.nn.relu(_dot(pd, P1_ref[...]) + p1b_ref[...])
    delta = jax.nn.relu(_dot(h1, P2_ref[...]) + p2b_ref[...])
    adst = adst_ref[...]
    al = (adst[None, :, :] - a_src + delta.reshape(K, tr, c)).reshape(K * tr, c)
    h2 = jax.nn.relu(_dot(al, A1_ref[...]) + a1b_ref[...])
    al = jax.nn.relu(_dot(h2, A2_ref[...]) + a2b_ref[...])
    a3 = al.reshape(K, tr, c)
    m = jnp.max(a3, axis=0)
    ex = jnp.exp(a3 - m[None, :, :])
    s = jnp.sum(ex, axis=0)
    wgt = ex / (s[None, :, :] + 1e-16)
    msg = wgt * (v + delta.reshape(K, tr, c))
    out = jnp.sum(msg, axis=0)
    out_ref[...] = jax.nn.relu(_dot(out, Wout_ref[...]) + bout_ref[...])


def _edge_block(G3, adst, pos_pad, p, tr=512):
    n = adst.shape[0]
    c = adst.shape[1]
    w = G3.shape[2]
    tr = min(tr, n)
    body = functools.partial(_edge_body, c=c, tr=tr)
    return pl.pallas_call(
        body,
        grid=(n // tr,),
        in_specs=[pl.BlockSpec((K, tr, w), lambda t: (0, t, 0)),
                  pl.BlockSpec((tr, c), lambda t: (t, 0)),
                  pl.BlockSpec((tr, 8), lambda t: (t, 0)),
                  pl.BlockSpec((3, 64), lambda t: (0, 0)),
                  pl.BlockSpec((1, 64), lambda t: (0, 0)),
                  pl.BlockSpec((64, c), lambda t: (0, 0)),
                  pl.BlockSpec((1, c), lambda t: (0, 0)),
                  pl.BlockSpec((c, 64), lambda t: (0, 0)),
                  pl.BlockSpec((1, 64), lambda t: (0, 0)),
                  pl.BlockSpec((64, c), lambda t: (0, 0)),
                  pl.BlockSpec((1, c), lambda t: (0, 0)),
                  pl.BlockSpec((c, c), lambda t: (0, 0)),
                  pl.BlockSpec((1, c), lambda t: (0, 0))],
        out_specs=pl.BlockSpec((tr, c), lambda t: (t, 0)),
        out_shape=jax.ShapeDtypeStruct((n, c), jnp.float32),
        interpret=_INTERPRET,
    )(G3, adst, pos_pad, p['P1'], p['p1b'][None, :], p['P2'], p['p2b'][None, :],
      p['A1'], p['a1b'][None, :], p['A2'], p['a2b'][None, :],
      p['Wout'], p['bout'][None, :])


def _td_bn_body(x_ref, W_ref, b_ref, o_ref):
    h = _dot(x_ref[...], W_ref[...]) + b_ref[...]
    c2 = h.shape[1]
    o_ref[:, 0:c2] = jax.nn.relu(_bn(h))
    if c2 < o_ref.shape[1]:
        o_ref[:, c2:] = jnp.zeros_like(o_ref[:, c2:])


def _td_bn(x, W, b):
    w = max(128, W.shape[1])
    return pl.pallas_call(
        _td_bn_body,
        out_shape=jax.ShapeDtypeStruct((x.shape[0], w), jnp.float32),
        interpret=_INTERPRET,
    )(x, W, b[None, :])


def _td_prep_body(Gh_ref, pos_ref, Win_ref, bin_ref, Wsrc_ref, Wdst_ref,
                  Wlin_ref, adst_ref, pack_ref):
    c_in = Win_ref.shape[0]
    x_sub = jnp.max(Gh_ref[...], axis=0)[:, 0:c_in]
    x1 = jax.nn.relu(_dot(x_sub, Win_ref[...]) + bin_ref[...])
    a_src = _dot(x1, Wsrc_ref[...])
    adst_ref[...] = _dot(x1, Wdst_ref[...])
    v = _dot(x1, Wlin_ref[...])
    c = a_src.shape[1]
    pack_ref[:, 0:c] = a_src
    pack_ref[:, c:2 * c] = v
    pack_ref[:, 2 * c:2 * c + 3] = pos_ref[...][:, 0:3]
    pack_ref[:, 2 * c + 3:] = jnp.zeros_like(pack_ref[:, 2 * c + 3:])


def _td_prep(Gh3, pos_pad, p):
    n1 = Gh3.shape[1]
    c = p['Win'].shape[0]
    w = _pack_width(c)
    return pl.pallas_call(
        _td_prep_body,
        out_shape=(jax.ShapeDtypeStruct((n1, c), jnp.float32),
                   jax.ShapeDtypeStruct((n1, w), jnp.float32)),
        interpret=_INTERPRET,
    )(Gh3, pos_pad, p['Win'], p['bin'][None, :], p['Wsrc'], p['Wdst'],
      p['Wlin'])


def _head_body(x_ref, W1_ref, b1_ref, W2_ref, b2_ref, o_ref):
    x = x_ref[...]
    g = jnp.sum(x, axis=0, keepdims=True) / x.shape[0]
    h = jax.nn.relu(_dot(g, W1_ref[...]) + b1_ref[...])
    o = _dot(h, W2_ref[...]) + b2_ref[...]
    m = jnp.max(o, axis=1, keepdims=True)
    e = jnp.exp(o - m)
    o_ref[...] = e / jnp.sum(e, axis=1, keepdims=True)


def _head(x, p):
    return pl.pallas_call(
        _head_body,
        out_shape=jax.ShapeDtypeStruct((1, NUM_CLASSES), jnp.float32),
        interpret=_INTERPRET,
    )(x, p['out_W1'], p['out_b1'][None, :], p['out_W2'], p['out_b2'][None, :])


def _pad_cols(a, w):
    return jnp.pad(a, ((0, 0), (0, w - a.shape[1])))


def _posT8(pos_pad):
    return jnp.transpose(pos_pad[:, 0:8])


def _jmajor(idx):
    return jnp.transpose(idx).reshape(-1)


def kernel(x, pos, batch, params):
    n = pos.shape[0]
    pos_pad = _pad_cols(pos, 8)
    posT = _posT8(pos_pad)
    pos3 = jnp.transpose(pos).reshape(3, n // 128, 128)

    adst0, P0 = _prep0(x, pos, params)
    ei0 = _knn(pos_pad, posT, exclude_self=True)
    G0 = _gather_rows(P0, _jmajor(ei0), 128).reshape(K, n, P0.shape[1])
    x0 = _edge_block(G0, adst0, pos_pad, params['blk0'])

    n1 = n // 4
    psub0 = _fps(pos3, n1)
    posT1 = _posT8(psub0)
    pos3_1 = jnp.transpose(psub0[:, 0:3]).reshape(3, n1 // 128, 128)
    nbr0 = _knn(psub0, posT, exclude_self=False)
    h0 = _td_bn(x0, params['td1_W'], params['td1_b'])
    Gh0 = _gather_rows(h0, _jmajor(nbr0), 128).reshape(K, n1, h0.shape[1])
    adst1, P1 = _td_prep(Gh0, psub0, params['blk1'])
    ei1 = _knn(psub0, posT1, exclude_self=True)
    G1 = _gather_rows(P1, _jmajor(ei1), 128).reshape(K, n1, P1.shape[1])
    x1 = _edge_block(G1, adst1, psub0, params['blk1'])

    n2 = n1 // 4
    psub1 = _fps(pos3_1, n2)
    posT2 = _posT8(psub1)
    nbr1 = _knn(psub1, posT1, exclude_self=False)
    h1 = _td_bn(x1, params['td2_W'], params['td2_b'])
    Gh1 = _gather_rows(h1, _jmajor(nbr1), 128).reshape(K, n2, h1.shape[1])
    adst2, P2 = _td_prep(Gh1, psub1, params['blk2'])
    ei2 = _knn(psub1, posT2, exclude_self=True)
    G2 = _gather_rows(P2, _jmajor(ei2), 128).reshape(K, n2, P2.shape[1])
    x2 = _edge_block(G2, adst2, psub1, params['blk2'])

    return _head(x2, params)


# speedup vs baseline: 1.0085x; 1.0085x over previous
import functools

import jax
import jax.numpy as jnp
from jax.experimental import pallas as pl
from jax.experimental.pallas import tpu as pltpu
from jax.experimental.pallas import tpu_sc as plsc

N = 8192
K = 16
DIMS = (32, 64, 128)
NUM_CLASSES = 40
INF = 1e10
BIG = 3e38

_INTERPRET = False


def _bn(h):
    mu = jnp.mean(h, axis=0, keepdims=True)
    var = jnp.mean((h - mu) ** 2, axis=0, keepdims=True)
    return (h - mu) / jnp.sqrt(var + 1e-5)


def _dot(a, b):
    return jax.lax.dot_general(a, b, (((1,), (0,)), ((), ())),
                               preferred_element_type=jnp.float32)


def _prep0_body(x_ref, pos_ref, inW_ref, inb_ref, Win_ref, bin_ref,
                Wsrc_ref, Wdst_ref, Wlin_ref, adst_ref, pack_ref):
    h = _dot(x_ref[...], inW_ref[...]) + inb_ref[...]
    x0 = jax.nn.relu(_bn(h))
    x1 = jax.nn.relu(_dot(x0, Win_ref[...]) + bin_ref[...])
    a_src = _dot(x1, Wsrc_ref[...])
    adst_ref[...] = _dot(x1, Wdst_ref[...])
    v = _dot(x1, Wlin_ref[...])
    c = a_src.shape[1]
    pack_ref[:, 0:c] = a_src
    pack_ref[:, c:2 * c] = v
    pack_ref[:, 2 * c:2 * c + 3] = pos_ref[...]
    pack_ref[:, 2 * c + 3:] = jnp.zeros_like(pack_ref[:, 2 * c + 3:])


def _pack_width(c):
    return -((2 * c + 3) // -128) * 128


def _prep0(x, pos, params):
    p = params['blk0']
    c = p['Win'].shape[0]
    w = _pack_width(c)
    return pl.pallas_call(
        _prep0_body,
        out_shape=(jax.ShapeDtypeStruct((x.shape[0], c), jnp.float32),
                   jax.ShapeDtypeStruct((x.shape[0], w), jnp.float32)),
        interpret=_INTERPRET,
    )(x, pos, params['in_W'], params['in_b'][None, :], p['Win'],
      p['bin'][None, :], p['Wsrc'], p['Wdst'], p['Wlin'])


def _knn_body(q_ref, sT_ref, out_ref, *, exclude_self, tr, ns):
    q = q_ref[...]
    sT = sT_ref[...]
    qs = jnp.sum(q * q, axis=1, keepdims=True)
    ss = jnp.sum(sT * sT, axis=0, keepdims=True)
    d = qs - 2.0 * _dot(q, sT) + ss
    colv = jax.lax.broadcasted_iota(jnp.int32, (tr, ns), 1)
    if exclude_self:
        row0 = pl.program_id(0) * tr
        rowv = row0 + jax.lax.broadcasted_iota(jnp.int32, (tr, ns), 0)
        d = jnp.where(colv == rowv, INF, d)
    for j in range(K):
        idx = jnp.argmin(d, axis=1).astype(jnp.int32)[:, None]
        out_ref[:, j:j + 1] = idx
        d = jnp.where(colv == idx, BIG, d)


def _knn(qpos_pad, sposT_pad, exclude_self, tr=256):
    nq = qpos_pad.shape[0]
    ns = sposT_pad.shape[1]
    body = functools.partial(_knn_body, exclude_self=exclude_self, tr=tr, ns=ns)
    return pl.pallas_call(
        body,
        grid=(nq // tr,),
        in_specs=[pl.BlockSpec((tr, 8), lambda t: (t, 0)),
                  pl.BlockSpec((8, ns), lambda t: (0, 0))],
        out_specs=pl.BlockSpec((tr, K), lambda t: (t, 0)),
        out_shape=jax.ShapeDtypeStruct((nq, K), jnp.int32),
        interpret=_INTERPRET,
    )(qpos_pad, sposT_pad)


def _fps_body(pos_ref, psub_ref, dist_ref, *, npts, rows):
    px = pos_ref[0]
    py = pos_ref[1]
    pz = pos_ref[2]
    lane3 = jax.lax.broadcasted_iota(jnp.int32, (1, 128), 1)
    rowi = jax.lax.broadcasted_iota(jnp.int32, (rows, 1), 0)

    def extract(r, lmsk):
        cx = jnp.sum(jnp.where(lmsk, pos_ref[0, pl.ds(r, 1), :], 0.0))
        cy = jnp.sum(jnp.where(lmsk, pos_ref[1, pl.ds(r, 1), :], 0.0))
        cz = jnp.sum(jnp.where(lmsk, pos_ref[2, pl.ds(r, 1), :], 0.0))
        return cx, cy, cz

    def store_row(i, cx, cy, cz):
        row = jnp.where(lane3 == 0, cx,
                        jnp.where(lane3 == 1, cy,
                                  jnp.where(lane3 == 2, cz, 0.0)))
        psub_ref[pl.ds(i, 1), :] = row[:, 0:psub_ref.shape[1]]

    cx, cy, cz = extract(0, lane3 == 0)
    d0 = (px - cx) ** 2 + (py - cy) ** 2 + (pz - cz) ** 2
    dist_ref[...] = d0
    store_row(0, cx, cy, cz)

    flati = (jax.lax.broadcasted_iota(jnp.int32, (rows, 128), 0) * 128
             + jax.lax.broadcasted_iota(jnp.int32, (rows, 128), 1))

    def body(i, _):
        dist = dist_ref[...]
        m = jnp.max(dist)
        idx = jnp.min(jnp.where(dist == m, flati, rows * 128))
        r = idx // 128
        lmsk = lane3 == (idx % 128)
        cx, cy, cz = extract(r, lmsk)
        store_row(i, cx, cy, cz)
        d = (px - cx) ** 2 + (py - cy) ** 2 + (pz - cz) ** 2
        dist_ref[...] = jnp.minimum(dist, d)
        return 0

    jax.lax.fori_loop(1, npts, body, 0)


def _fps(pos3, npts):
    rows = pos3.shape[1]
    body = functools.partial(_fps_body, npts=npts, rows=rows)
    return pl.pallas_call(
        body,
        out_shape=jax.ShapeDtypeStruct((npts, 8), jnp.float32),
        scratch_shapes=[pltpu.VMEM((rows, 128), jnp.float32)],
        interpret=_INTERPRET,
    )(pos3)


def _sc_gather_rows(data, idx_flat, window):
    e = idx_flat.shape[0]
    w = data.shape[1]
    idx2 = idx_flat.reshape(1, e)
    mesh = plsc.VectorSubcoreMesh(core_axis_name="c", subcore_axis_name="s")

    @functools.partial(
        pl.kernel,
        out_type=jax.ShapeDtypeStruct((e, w), data.dtype),
        mesh=mesh)
    def kern(x_hbm, i_hbm, o_hbm):
        def body(i_vmem, o_vmem):
            pltpu.sync_copy(x_hbm.at[i_vmem.at[0]], o_vmem)

        pltpu.emit_pipeline(
            body,
            grid=(e // window,),
            in_specs=[pl.BlockSpec((1, window), lambda i: (0, i))],
            out_specs=[pl.BlockSpec((window, w), lambda i: (i, 0))],
            core_axis_name=("c", "s"),
            dimension_semantics=(pltpu.PARALLEL,),
        )(i_hbm, o_hbm)

    return kern(data, idx2)


def _gather_rows(data, idx_flat, window):
    return _sc_gather_rows(data, idx_flat, window)


def _edge_body(G_ref, adst_ref, pos_ref, P1_ref, p1b_ref, P2_ref, p2b_ref,
               A1_ref, a1b_ref, A2_ref, a2b_ref, Wout_ref, bout_ref, out_ref,
               *, c, tr):
    G = G_ref[...]
    a_src = G[:, :, 0:c]
    v = G[:, :, c:2 * c]
    posj = G[:, :, 2 * c:2 * c + 3]
    posi = pos_ref[...][:, 0:3]
    pd = (posi[None, :, :] - posj).reshape(K * tr, 3)
    h1 = jax.nn.relu(_dot(pd, P1_ref[...]) + p1b_ref[...])
    delta = jax.nn.relu(_dot(h1, P2_ref[...]) + p2b_ref[...])
    adst = adst_ref[...]
    al = (adst[None, :, :] - a_src + delta.reshape(K, tr, c)).reshape(K * tr, c)
    h2 = jax.nn.relu(_dot(al, A1_ref[...]) + a1b_ref[...])
    al = jax.nn.relu(_dot(h2, A2_ref[...]) + a2b_ref[...])
    a3 = al.reshape(K, tr, c)
    m = jnp.max(a3, axis=0)
    ex = jnp.exp(a3 - m[None, :, :])
    s = jnp.sum(ex, axis=0)
    wgt = ex / (s[None, :, :] + 1e-16)
    msg = wgt * (v + delta.reshape(K, tr, c))
    out = jnp.sum(msg, axis=0)
    out_ref[...] = jax.nn.relu(_dot(out, Wout_ref[...]) + bout_ref[...])


def _edge_block(G3, adst, pos_pad, p, tr=512):
    n = adst.shape[0]
    c = adst.shape[1]
    w = G3.shape[2]
    tr = min(tr, n)
    body = functools.partial(_edge_body, c=c, tr=tr)
    return pl.pallas_call(
        body,
        grid=(n // tr,),
        in_specs=[pl.BlockSpec((K, tr, w), lambda t: (0, t, 0)),
                  pl.BlockSpec((tr, c), lambda t: (t, 0)),
                  pl.BlockSpec((tr, 8), lambda t: (t, 0)),
                  pl.BlockSpec((3, 64), lambda t: (0, 0)),
                  pl.BlockSpec((1, 64), lambda t: (0, 0)),
                  pl.BlockSpec((64, c), lambda t: (0, 0)),
                  pl.BlockSpec((1, c), lambda t: (0, 0)),
                  pl.BlockSpec((c, 64), lambda t: (0, 0)),
                  pl.BlockSpec((1, 64), lambda t: (0, 0)),
                  pl.BlockSpec((64, c), lambda t: (0, 0)),
                  pl.BlockSpec((1, c), lambda t: (0, 0)),
                  pl.BlockSpec((c, c), lambda t: (0, 0)),
                  pl.BlockSpec((1, c), lambda t: (0, 0))],
        out_specs=pl.BlockSpec((tr, c), lambda t: (t, 0)),
        out_shape=jax.ShapeDtypeStruct((n, c), jnp.float32),
        interpret=_INTERPRET,
    )(G3, adst, pos_pad, p['P1'], p['p1b'][None, :], p['P2'], p['p2b'][None, :],
      p['A1'], p['a1b'][None, :], p['A2'], p['a2b'][None, :],
      p['Wout'], p['bout'][None, :])


def _td_bn_body(x_ref, W_ref, b_ref, o_ref):
    h = _dot(x_ref[...], W_ref[...]) + b_ref[...]
    c2 = h.shape[1]
    o_ref[:, 0:c2] = jax.nn.relu(_bn(h))
    if c2 < o_ref.shape[1]:
        o_ref[:, c2:] = jnp.zeros_like(o_ref[:, c2:])


def _td_bn(x, W, b):
    w = max(128, W.shape[1])
    return pl.pallas_call(
        _td_bn_body,
        out_shape=jax.ShapeDtypeStruct((x.shape[0], w), jnp.float32),
        interpret=_INTERPRET,
    )(x, W, b[None, :])


def _td_prep_body(Gh_ref, pos_ref, Win_ref, bin_ref, Wsrc_ref, Wdst_ref,
                  Wlin_ref, adst_ref, pack_ref):
    c_in = Win_ref.shape[0]
    x_sub = jnp.max(Gh_ref[...], axis=0)[:, 0:c_in]
    x1 = jax.nn.relu(_dot(x_sub, Win_ref[...]) + bin_ref[...])
    a_src = _dot(x1, Wsrc_ref[...])
    adst_ref[...] = _dot(x1, Wdst_ref[...])
    v = _dot(x1, Wlin_ref[...])
    c = a_src.shape[1]
    pack_ref[:, 0:c] = a_src
    pack_ref[:, c:2 * c] = v
    pack_ref[:, 2 * c:2 * c + 3] = pos_ref[...][:, 0:3]
    pack_ref[:, 2 * c + 3:] = jnp.zeros_like(pack_ref[:, 2 * c + 3:])


def _td_prep(Gh3, pos_pad, p):
    n1 = Gh3.shape[1]
    c = p['Win'].shape[0]
    w = _pack_width(c)
    return pl.pallas_call(
        _td_prep_body,
        out_shape=(jax.ShapeDtypeStruct((n1, c), jnp.float32),
                   jax.ShapeDtypeStruct((n1, w), jnp.float32)),
        interpret=_INTERPRET,
    )(Gh3, pos_pad, p['Win'], p['bin'][None, :], p['Wsrc'], p['Wdst'],
      p['Wlin'])


def _head_body(x_ref, W1_ref, b1_ref, W2_ref, b2_ref, o_ref):
    x = x_ref[...]
    g = jnp.sum(x, axis=0, keepdims=True) / x.shape[0]
    h = jax.nn.relu(_dot(g, W1_ref[...]) + b1_ref[...])
    o = _dot(h, W2_ref[...]) + b2_ref[...]
    m = jnp.max(o, axis=1, keepdims=True)
    e = jnp.exp(o - m)
    o_ref[...] = e / jnp.sum(e, axis=1, keepdims=True)


def _head(x, p):
    return pl.pallas_call(
        _head_body,
        out_shape=jax.ShapeDtypeStruct((1, NUM_CLASSES), jnp.float32),
        interpret=_INTERPRET,
    )(x, p['out_W1'], p['out_b1'][None, :], p['out_W2'], p['out_b2'][None, :])


def _pad_cols(a, w):
    return jnp.pad(a, ((0, 0), (0, w - a.shape[1])))


def _posT8(pos_pad):
    return jnp.transpose(pos_pad[:, 0:8])


def _jmajor(idx):
    return jnp.transpose(idx).reshape(-1)


def kernel(x, pos, batch, params):
    n = pos.shape[0]
    pos_pad = _pad_cols(pos, 8)
    posT = _posT8(pos_pad)
    pos3 = jnp.transpose(pos).reshape(3, n // 128, 128)

    adst0, P0 = _prep0(x, pos, params)
    ei0 = _knn(pos_pad, posT, exclude_self=True)
    G0 = _gather_rows(P0, _jmajor(ei0), 128).reshape(K, n, P0.shape[1])
    x0 = _edge_block(G0, adst0, pos_pad, params['blk0'])

    n1 = n // 4
    psub0 = _fps(pos3, n1)
    posT1 = _posT8(psub0)
    pos3_1 = jnp.transpose(psub0[:, 0:3]).reshape(3, n1 // 128, 128)
    nbr0 = _knn(psub0, posT, exclude_self=False)
    h0 = _td_bn(x0, params['td1_W'], params['td1_b'])
    Gh0 = _gather_rows(h0, _jmajor(nbr0), 128).reshape(K, n1, h0.shape[1])
    adst1, P1 = _td_prep(Gh0, psub0, params['blk1'])
    ei1 = _knn(psub0, posT1, exclude_self=True)
    G1 = _gather_rows(P1, _jmajor(ei1), 128).reshape(K, n1, P1.shape[1])
    x1 = _edge_block(G1, adst1, psub0, params['blk1'])

    n2 = n1 // 4
    psub1 = _fps(pos3_1, n2)
    posT2 = _posT8(psub1)
    nbr1 = _knn(psub1, posT1, exclude_self=False)
    h1 = _td_bn(x1, params['td2_W'], params['td2_b'])
    Gh1 = _gather_rows(h1, _jmajor(nbr1), 128).reshape(K, n2, h1.shape[1])
    adst2, P2 = _td_prep(Gh1, psub1, params['blk2'])
    ei2 = _knn(psub1, posT2, exclude_self=True)
    G2 = _gather_rows(P2, _jmajor(ei2), 128).reshape(K, n2, P2.shape[1])
    x2 = _edge_block(G2, adst2, psub1, params['blk2'])

    return _head(x2, params)


# speedup vs baseline: 1.0092x; 1.0007x over previous
import functools

import jax
import jax.numpy as jnp
from jax.experimental import pallas as pl
from jax.experimental.pallas import tpu as pltpu
from jax.experimental.pallas import tpu_sc as plsc

K = 16
NUM_CLASSES = 40
INF = 1e10
BIG = 3e38


def _bn(h):
    mu = jnp.mean(h, axis=0, keepdims=True)
    var = jnp.mean((h - mu) ** 2, axis=0, keepdims=True)
    return (h - mu) / jnp.sqrt(var + 1e-5)


def _dot(a, b):
    return jax.lax.dot_general(a, b, (((1,), (0,)), ((), ())),
                               preferred_element_type=jnp.float32)


def _prep0_body(x_ref, pos_ref, inW_ref, inb_ref, Win_ref, bin_ref,
                Wsrc_ref, Wdst_ref, Wlin_ref, adst_ref, pack_ref):
    h = _dot(x_ref[...], inW_ref[...]) + inb_ref[...]
    x0 = jax.nn.relu(_bn(h))
    x1 = jax.nn.relu(_dot(x0, Win_ref[...]) + bin_ref[...])
    a_src = _dot(x1, Wsrc_ref[...])
    adst_ref[...] = _dot(x1, Wdst_ref[...])
    v = _dot(x1, Wlin_ref[...])
    c = a_src.shape[1]
    pack_ref[:, 0:c] = a_src
    pack_ref[:, c:2 * c] = v
    pack_ref[:, 2 * c:2 * c + 3] = pos_ref[...]
    pack_ref[:, 2 * c + 3:] = jnp.zeros_like(pack_ref[:, 2 * c + 3:])


def _pack_width(c):
    return -((2 * c + 3) // -128) * 128


def _prep0(x, pos, params):
    p = params['blk0']
    c = p['Win'].shape[0]
    w = _pack_width(c)
    return pl.pallas_call(
        _prep0_body,
        out_shape=(jax.ShapeDtypeStruct((x.shape[0], c), jnp.float32),
                   jax.ShapeDtypeStruct((x.shape[0], w), jnp.float32)),
    )(x, pos, params['in_W'], params['in_b'][None, :], p['Win'],
      p['bin'][None, :], p['Wsrc'], p['Wdst'], p['Wlin'])


def _knn_body(q_ref, sT_ref, out_ref, *, exclude_self, tr, ns):
    q = q_ref[...]
    sT = sT_ref[...]
    qs = jnp.sum(q * q, axis=1, keepdims=True)
    ss = jnp.sum(sT * sT, axis=0, keepdims=True)
    d = qs - 2.0 * _dot(q, sT) + ss
    colv = jax.lax.broadcasted_iota(jnp.int32, (tr, ns), 1)
    if exclude_self:
        row0 = pl.program_id(0) * tr
        rowv = row0 + jax.lax.broadcasted_iota(jnp.int32, (tr, ns), 0)
        d = jnp.where(colv == rowv, INF, d)
    for j in range(K):
        idx = jnp.argmin(d, axis=1).astype(jnp.int32)[:, None]
        out_ref[:, j:j + 1] = idx
        d = jnp.where(colv == idx, BIG, d)


def _knn(qpos_pad, sposT_pad, exclude_self, tr=256):
    nq = qpos_pad.shape[0]
    ns = sposT_pad.shape[1]
    body = functools.partial(_knn_body, exclude_self=exclude_self, tr=tr, ns=ns)
    return pl.pallas_call(
        body,
        grid=(nq // tr,),
        in_specs=[pl.BlockSpec((tr, 8), lambda t: (t, 0)),
                  pl.BlockSpec((8, ns), lambda t: (0, 0))],
        out_specs=pl.BlockSpec((tr, K), lambda t: (t, 0)),
        out_shape=jax.ShapeDtypeStruct((nq, K), jnp.int32),
    )(qpos_pad, sposT_pad)


def _fps_body(pos_ref, psub_ref, dist_ref, *, npts, rows):
    px = pos_ref[0]
    py = pos_ref[1]
    pz = pos_ref[2]
    lane3 = jax.lax.broadcasted_iota(jnp.int32, (1, 128), 1)

    def extract(r, lmsk):
        cx = jnp.sum(jnp.where(lmsk, pos_ref[0, pl.ds(r, 1), :], 0.0))
        cy = jnp.sum(jnp.where(lmsk, pos_ref[1, pl.ds(r, 1), :], 0.0))
        cz = jnp.sum(jnp.where(lmsk, pos_ref[2, pl.ds(r, 1), :], 0.0))
        return cx, cy, cz

    def store_row(i, cx, cy, cz):
        row = jnp.where(lane3 == 0, cx,
                        jnp.where(lane3 == 1, cy,
                                  jnp.where(lane3 == 2, cz, 0.0)))
        psub_ref[pl.ds(i, 1), :] = row[:, 0:psub_ref.shape[1]]

    cx, cy, cz = extract(0, lane3 == 0)
    d0 = (px - cx) ** 2 + (py - cy) ** 2 + (pz - cz) ** 2
    dist_ref[...] = d0
    store_row(0, cx, cy, cz)

    flati = (jax.lax.broadcasted_iota(jnp.int32, (rows, 128), 0) * 128
             + jax.lax.broadcasted_iota(jnp.int32, (rows, 128), 1))

    def body(i, _):
        dist = dist_ref[...]
        m = jnp.max(dist)
        idx = jnp.min(jnp.where(dist == m, flati, rows * 128))
        r = idx // 128
        lmsk = lane3 == (idx % 128)
        cx, cy, cz = extract(r, lmsk)
        store_row(i, cx, cy, cz)
        d = (px - cx) ** 2 + (py - cy) ** 2 + (pz - cz) ** 2
        dist_ref[...] = jnp.minimum(dist, d)
        return 0

    jax.lax.fori_loop(1, npts, body, 0)


def _fps(pos3, npts):
    rows = pos3.shape[1]
    body = functools.partial(_fps_body, npts=npts, rows=rows)
    return pl.pallas_call(
        body,
        out_shape=jax.ShapeDtypeStruct((npts, 8), jnp.float32),
        scratch_shapes=[pltpu.VMEM((rows, 128), jnp.float32)],
    )(pos3)


def _sc_gather_rows(data, idx_flat, window):
    e = idx_flat.shape[0]
    w = data.shape[1]
    idx2 = idx_flat.reshape(1, e)
    mesh = plsc.VectorSubcoreMesh(core_axis_name="c", subcore_axis_name="s")

    @functools.partial(
        pl.kernel,
        out_type=jax.ShapeDtypeStruct((e, w), data.dtype),
        mesh=mesh)
    def kern(x_hbm, i_hbm, o_hbm):
        def body(i_vmem, o_vmem):
            pltpu.sync_copy(x_hbm.at[i_vmem.at[0]], o_vmem)

        pltpu.emit_pipeline(
            body,
            grid=(e // window,),
            in_specs=[pl.BlockSpec((1, window), lambda i: (0, i))],
            out_specs=[pl.BlockSpec((window, w), lambda i: (i, 0))],
            core_axis_name=("c", "s"),
            dimension_semantics=(pltpu.PARALLEL,),
        )(i_hbm, o_hbm)

    return kern(data, idx2)


def _gather_rows(data, idx_flat, window):
    return _sc_gather_rows(data, idx_flat, window)


def _edge_body(G_ref, adst_ref, pos_ref, P1_ref, p1b_ref, P2_ref, p2b_ref,
               A1_ref, a1b_ref, A2_ref, a2b_ref, Wout_ref, bout_ref, out_ref,
               *, c, tr):
    G = G_ref[...]
    a_src = G[:, :, 0:c]
    v = G[:, :, c:2 * c]
    posj = G[:, :, 2 * c:2 * c + 3]
    posi = pos_ref[...][:, 0:3]
    pd = (posi[None, :, :] - posj).reshape(K * tr, 3)
    h1 = jax.nn.relu(_dot(pd, P1_ref[...]) + p1b_ref[...])
    delta = jax.nn.relu(_dot(h1, P2_ref[...]) + p2b_ref[...])
    adst = adst_ref[...]
    al = (adst[None, :, :] - a_src + delta.reshape(K, tr, c)).reshape(K * tr, c)
    h2 = jax.nn.relu(_dot(al, A1_ref[...]) + a1b_ref[...])
    al = jax.nn.relu(_dot(h2, A2_ref[...]) + a2b_ref[...])
    a3 = al.reshape(K, tr, c)
    m = jnp.max(a3, axis=0)
    ex = jnp.exp(a3 - m[None, :, :])
    s = jnp.sum(ex, axis=0)
    wgt = ex / (s[None, :, :] + 1e-16)
    msg = wgt * (v + delta.reshape(K, tr, c))
    out = jnp.sum(msg, axis=0)
    out_ref[...] = jax.nn.relu(_dot(out, Wout_ref[...]) + bout_ref[...])


def _edge_block(G3, adst, pos_pad, p, tr=512):
    n = adst.shape[0]
    c = adst.shape[1]
    w = G3.shape[2]
    tr = min(tr, n)
    body = functools.partial(_edge_body, c=c, tr=tr)
    return pl.pallas_call(
        body,
        grid=(n // tr,),
        in_specs=[pl.BlockSpec((K, tr, w), lambda t: (0, t, 0)),
                  pl.BlockSpec((tr, c), lambda t: (t, 0)),
                  pl.BlockSpec((tr, 8), lambda t: (t, 0)),
                  pl.BlockSpec((3, 64), lambda t: (0, 0)),
                  pl.BlockSpec((1, 64), lambda t: (0, 0)),
                  pl.BlockSpec((64, c), lambda t: (0, 0)),
                  pl.BlockSpec((1, c), lambda t: (0, 0)),
                  pl.BlockSpec((c, 64), lambda t: (0, 0)),
                  pl.BlockSpec((1, 64), lambda t: (0, 0)),
                  pl.BlockSpec((64, c), lambda t: (0, 0)),
                  pl.BlockSpec((1, c), lambda t: (0, 0)),
                  pl.BlockSpec((c, c), lambda t: (0, 0)),
                  pl.BlockSpec((1, c), lambda t: (0, 0))],
        out_specs=pl.BlockSpec((tr, c), lambda t: (t, 0)),
        out_shape=jax.ShapeDtypeStruct((n, c), jnp.float32),
    )(G3, adst, pos_pad, p['P1'], p['p1b'][None, :], p['P2'], p['p2b'][None, :],
      p['A1'], p['a1b'][None, :], p['A2'], p['a2b'][None, :],
      p['Wout'], p['bout'][None, :])


def _td_bn_body(x_ref, W_ref, b_ref, o_ref):
    h = _dot(x_ref[...], W_ref[...]) + b_ref[...]
    c2 = h.shape[1]
    o_ref[:, 0:c2] = jax.nn.relu(_bn(h))
    if c2 < o_ref.shape[1]:
        o_ref[:, c2:] = jnp.zeros_like(o_ref[:, c2:])


def _td_bn(x, W, b):
    w = max(128, W.shape[1])
    return pl.pallas_call(
        _td_bn_body,
        out_shape=jax.ShapeDtypeStruct((x.shape[0], w), jnp.float32),
    )(x, W, b[None, :])


def _td_prep_body(Gh_ref, pos_ref, Win_ref, bin_ref, Wsrc_ref, Wdst_ref,
                  Wlin_ref, adst_ref, pack_ref):
    c_in = Win_ref.shape[0]
    x_sub = jnp.max(Gh_ref[...], axis=0)[:, 0:c_in]
    x1 = jax.nn.relu(_dot(x_sub, Win_ref[...]) + bin_ref[...])
    a_src = _dot(x1, Wsrc_ref[...])
    adst_ref[...] = _dot(x1, Wdst_ref[...])
    v = _dot(x1, Wlin_ref[...])
    c = a_src.shape[1]
    pack_ref[:, 0:c] = a_src
    pack_ref[:, c:2 * c] = v
    pack_ref[:, 2 * c:2 * c + 3] = pos_ref[...][:, 0:3]
    pack_ref[:, 2 * c + 3:] = jnp.zeros_like(pack_ref[:, 2 * c + 3:])


def _td_prep(Gh3, pos_pad, p):
    n1 = Gh3.shape[1]
    c = p['Win'].shape[0]
    w = _pack_width(c)
    return pl.pallas_call(
        _td_prep_body,
        out_shape=(jax.ShapeDtypeStruct((n1, c), jnp.float32),
                   jax.ShapeDtypeStruct((n1, w), jnp.float32)),
    )(Gh3, pos_pad, p['Win'], p['bin'][None, :], p['Wsrc'], p['Wdst'],
      p['Wlin'])


def _head_body(x_ref, W1_ref, b1_ref, W2_ref, b2_ref, o_ref):
    x = x_ref[...]
    g = jnp.sum(x, axis=0, keepdims=True) / x.shape[0]
    h = jax.nn.relu(_dot(g, W1_ref[...]) + b1_ref[...])
    o = _dot(h, W2_ref[...]) + b2_ref[...]
    m = jnp.max(o, axis=1, keepdims=True)
    e = jnp.exp(o - m)
    o_ref[...] = e / jnp.sum(e, axis=1, keepdims=True)


def _head(x, p):
    return pl.pallas_call(
        _head_body,
        out_shape=jax.ShapeDtypeStruct((1, NUM_CLASSES), jnp.float32),
    )(x, p['out_W1'], p['out_b1'][None, :], p['out_W2'], p['out_b2'][None, :])


def _pad_cols(a, w):
    return jnp.pad(a, ((0, 0), (0, w - a.shape[1])))


def _posT8(pos_pad):
    return jnp.transpose(pos_pad[:, 0:8])


def _jmajor(idx):
    return jnp.transpose(idx).reshape(-1)


def kernel(x, pos, batch, params):
    n = pos.shape[0]
    pos_pad = _pad_cols(pos, 8)
    posT = _posT8(pos_pad)
    pos3 = jnp.transpose(pos).reshape(3, n // 128, 128)

    adst0, P0 = _prep0(x, pos, params)
    ei0 = _knn(pos_pad, posT, exclude_self=True)
    G0 = _gather_rows(P0, _jmajor(ei0), 128).reshape(K, n, P0.shape[1])
    x0 = _edge_block(G0, adst0, pos_pad, params['blk0'])

    n1 = n // 4
    psub0 = _fps(pos3, n1)
    posT1 = _posT8(psub0)
    pos3_1 = jnp.transpose(psub0[:, 0:3]).reshape(3, n1 // 128, 128)
    nbr0 = _knn(psub0, posT, exclude_self=False)
    h0 = _td_bn(x0, params['td1_W'], params['td1_b'])
    Gh0 = _gather_rows(h0, _jmajor(nbr0), 128).reshape(K, n1, h0.shape[1])
    adst1, P1 = _td_prep(Gh0, psub0, params['blk1'])
    ei1 = _knn(psub0, posT1, exclude_self=True)
    G1 = _gather_rows(P1, _jmajor(ei1), 128).reshape(K, n1, P1.shape[1])
    x1 = _edge_block(G1, adst1, psub0, params['blk1'])

    n2 = n1 // 4
    psub1 = _fps(pos3_1, n2)
    posT2 = _posT8(psub1)
    nbr1 = _knn(psub1, posT1, exclude_self=False)
    h1 = _td_bn(x1, params['td2_W'], params['td2_b'])
    Gh1 = _gather_rows(h1, _jmajor(nbr1), 128).reshape(K, n2, h1.shape[1])
    adst2, P2 = _td_prep(Gh1, psub1, params['blk2'])
    ei2 = _knn(psub1, posT2, exclude_self=True)
    G2 = _gather_rows(P2, _jmajor(ei2), 128).reshape(K, n2, P2.shape[1])
    x2 = _edge_block(G2, adst2, psub1, params['blk2'])

    return _head(x2, params)
